# baseline (speedup 1.0000x reference)
import functools

import jax
import jax.numpy as jnp
from jax import lax
from jax.experimental import pallas as pl
from jax.experimental.pallas import tpu as pltpu
from jax.experimental.pallas import tpu_sc as plsc

VOCAB = 1000000
EMB = 64
BATCH = 4096
SEQ = 200

N = BATCH * SEQ
NW = 32
PER_W = N // NW
CHUNK = 128
NCHUNK = PER_W // CHUNK

_mesh = plsc.VectorSubcoreMesh(core_axis_name="c", subcore_axis_name="s")


@functools.partial(
    pl.kernel,
    out_type=jax.ShapeDtypeStruct((N, EMB), jnp.float32),
    mesh=_mesh,
    scratch_types=[
        pltpu.VMEM((NCHUNK, CHUNK), jnp.int32),
        pltpu.VMEM((CHUNK, EMB), jnp.float32),
        pltpu.SemaphoreType.DMA,
    ],
    compiler_params=pltpu.CompilerParams(use_tc_tiling_on_sc=False),
)
def _embed_gather(idx_hbm, table_hbm, out_hbm, idx_v, rows_v, sem):
    cid = lax.axis_index("c")
    sid = lax.axis_index("s")
    wid = sid * 2 + cid
    base = wid * PER_W

    pltpu.sync_copy(idx_hbm.at[wid], idx_v)

    def body(j, carry):
        pltpu.async_copy(table_hbm.at[idx_v.at[j]], rows_v, sem).wait()
        pltpu.sync_copy(rows_v, out_hbm.at[pl.ds(base + j * CHUNK, CHUNK)])
        return carry

    lax.fori_loop(0, NCHUNK, body, 0)


def kernel(inp, hidden, embed_weight):
    del hidden
    idx = inp.astype(jnp.int32).reshape(NW, NCHUNK, CHUNK)
    out = _embed_gather(idx, embed_weight)
    return out.reshape(BATCH, SEQ, EMB)


# speedup vs baseline: 1.1134x; 1.1134x over previous
import functools

import jax
import jax.numpy as jnp
from jax import lax
from jax.experimental import pallas as pl
from jax.experimental.pallas import tpu as pltpu
from jax.experimental.pallas import tpu_sc as plsc

VOCAB = 1000000
EMB = 64
BATCH = 4096
SEQ = 200

N = BATCH * SEQ
NW = 32
PER_W = N // NW
CHUNK = 128
NCHUNK = PER_W // CHUNK

K = 5
SUPER = K * CHUNK
NSUPER = NCHUNK // K
NPAIR = NSUPER // 2

_mesh = plsc.VectorSubcoreMesh(core_axis_name="c", subcore_axis_name="s")


@functools.partial(
    pl.kernel,
    out_type=jax.ShapeDtypeStruct((N, EMB), jnp.float32),
    mesh=_mesh,
    scratch_types=[
        pltpu.VMEM((NCHUNK, CHUNK), jnp.int32),
        pltpu.VMEM((SUPER, EMB), jnp.float32),
        pltpu.VMEM((SUPER, EMB), jnp.float32),
        pltpu.SemaphoreType.DMA,
        pltpu.SemaphoreType.DMA,
        pltpu.SemaphoreType.DMA,
        pltpu.SemaphoreType.DMA,
    ],
    compiler_params=pltpu.CompilerParams(use_tc_tiling_on_sc=False),
)
def _embed_gather(idx_hbm, table_hbm, out_hbm, idx_v, buf0, buf1,
                  gsem0, gsem1, ssem0, ssem1):
    cid = lax.axis_index("c")
    sid = lax.axis_index("s")
    wid = sid * 2 + cid
    base = wid * PER_W

    pltpu.sync_copy(idx_hbm.at[wid], idx_v)

    bufs = (buf0, buf1)
    gsems = (gsem0, gsem1)
    ssems = (ssem0, ssem1)

    def body(t, carry):
        for b in range(2):
            s = 2 * t + b
            buf, gsem, ssem = bufs[b], gsems[b], ssems[b]

            @pl.when(t > 0)
            def _():
                pltpu.make_async_copy(
                    buf, out_hbm.at[pl.ds(base + s * SUPER, SUPER)], ssem
                ).wait()

            handles = [
                pltpu.async_copy(
                    table_hbm.at[idx_v.at[s * K + k]],
                    buf.at[pl.ds(k * CHUNK, CHUNK)],
                    gsem,
                )
                for k in range(K)
            ]
            for h in handles:
                h.wait()

            pltpu.async_copy(buf, out_hbm.at[pl.ds(base + s * SUPER, SUPER)], ssem)
        return carry

    lax.fori_loop(0, NPAIR, body, 0)

    for b in range(2):
        pltpu.make_async_copy(
            bufs[b], out_hbm.at[pl.ds(base, SUPER)], ssems[b]
        ).wait()


def kernel(inp, hidden, embed_weight):
    del hidden
    idx = inp.astype(jnp.int32).reshape(NW, NCHUNK, CHUNK)
    out = _embed_gather(idx, embed_weight)
    return out.reshape(BATCH, SEQ, EMB)
